# baseline (speedup 1.0000x reference)
import functools

import jax
import jax.numpy as jnp
from jax import lax
from jax.experimental import pallas as pl
from jax.experimental.pallas import tpu as pltpu
from jax.experimental.pallas import tpu_sc as plsc

NC = 2
NS = 16
NW = NC * NS
L = 16

B = 16384
D = 32
BPW = B // NW
CHUNK = 128
NCH = BPW // CHUNK


def _mf_body(user_r, item_r, ue_r, ie_r, ub_r, ib_r, out_r,
             idx_u, idx_i, rows_u, rows_i, bu, bi, preds, sem):
  wid = lax.axis_index("s") * NC + lax.axis_index("c")

  pltpu.sync_copy(user_r.at[wid], idx_u)
  pltpu.sync_copy(item_r.at[wid], idx_i)

  copies = []
  for c in range(NCH):
    copies.append(pltpu.async_copy(ue_r.at[idx_u.at[c]], rows_u.at[c], sem))
    copies.append(pltpu.async_copy(ie_r.at[idx_i.at[c]], rows_i.at[c], sem))
    copies.append(pltpu.async_copy(ub_r.at[idx_u.at[c]], bu.at[c], sem))
    copies.append(pltpu.async_copy(ib_r.at[idx_i.at[c]], bi.at[c], sem))
  for cp in copies:
    cp.wait()

  lane = lax.iota(jnp.int32, L)
  for c in range(NCH):
    def group_body(g, _, c=c):
      acc = jnp.zeros((L,), jnp.float32)
      for u in range(L):
        r = g * L + u
        p = (rows_u[c, r, pl.ds(0, L)] * rows_i[c, r, pl.ds(0, L)]
             + rows_u[c, r, pl.ds(L, L)] * rows_i[c, r, pl.ds(L, L)])
        acc = jnp.where(lane == u, jnp.sum(p), acc)
      x = acc + bu[c, pl.ds(g * L, L)] + bi[c, pl.ds(g * L, L)]
      preds[pl.ds(c * CHUNK + g * L, L)] = 1.0 / (1.0 + jnp.exp(-x))
      return 0
    lax.fori_loop(0, CHUNK // L, group_body, 0)

  pltpu.sync_copy(preds, out_r.at[wid])


@functools.partial(jax.jit, donate_argnums=())
def kernel(user, item, user_e, item_e, user_b, item_b):
  user = user.astype(jnp.int32).reshape(NW, NCH, CHUNK)
  item = item.astype(jnp.int32).reshape(NW, NCH, CHUNK)
  ub = user_b.reshape(-1)
  ib = item_b.reshape(-1)

  mesh = plsc.VectorSubcoreMesh(core_axis_name="c", subcore_axis_name="s")
  run = pl.kernel(
      _mf_body,
      out_type=jax.ShapeDtypeStruct((NW, BPW), jnp.float32),
      mesh=mesh,
      compiler_params=pltpu.CompilerParams(
          needs_layout_passes=False, use_tc_tiling_on_sc=False),
      scratch_types=[
          pltpu.VMEM((NCH, CHUNK), jnp.int32),
          pltpu.VMEM((NCH, CHUNK), jnp.int32),
          pltpu.VMEM((NCH, CHUNK, D), jnp.float32),
          pltpu.VMEM((NCH, CHUNK, D), jnp.float32),
          pltpu.VMEM((NCH, CHUNK), jnp.float32),
          pltpu.VMEM((NCH, CHUNK), jnp.float32),
          pltpu.VMEM((BPW,), jnp.float32),
          pltpu.SemaphoreType.DMA,
      ],
  )
  out = run(user, item, user_e, item_e, ub, ib)
  return out.reshape(B)
